# baseline (speedup 1.0000x reference)
import functools

import jax
import jax.numpy as jnp
from jax import lax
from jax.experimental import pallas as pl
from jax.experimental.pallas import tpu as pltpu
from jax.experimental.pallas import tpu_sc as plsc

N_USER_C = 50000
N_ITEM_C = 50000
N_C = N_USER_C + N_ITEM_C
HALF_N = N_C // 2
E_C = 1600000
D_C = 32
B_C = 4096
LAYERS_C = 3

NC = 2
NS = 16
NW = NC * NS

GSZ = 128

PT_GROUPS = 400
E_PAD = NW * PT_GROUPS * GSZ
G_TOTAL = E_PAD // GSZ
P_G = 50
PASSES = PT_GROUPS // P_G
STG_E = P_G * GSZ
CAPG = 440
CAP_E = CAPG * GSZ

CHUNK_G = 16
CHUNK_E = CHUNK_G * GSZ
MAXC = (CAPG + CHUNK_G - 1) // CHUNK_G

STRIPE = 3128
ZB = 64

_mesh = plsc.VectorSubcoreMesh(core_axis_name="c", subcore_axis_name="s")
_cparams = pltpu.CompilerParams(use_tc_tiling_on_sc=False)
_cparams_nl = pltpu.CompilerParams(
    use_tc_tiling_on_sc=False, needs_layout_passes=False)


def _iota16():
    return lax.broadcasted_iota(jnp.int32, (16,), 0)


_GD = lax.GatherDimensionNumbers(
    offset_dims=(), collapsed_slice_dims=(0,), start_index_map=(0,))


@functools.partial(
    pl.kernel,
    out_type=[
        jax.ShapeDtypeStruct((NC, NW, CAPG, GSZ), jnp.int32),
        jax.ShapeDtypeStruct((NC, NW, CAPG, GSZ), jnp.int32),
        jax.ShapeDtypeStruct((NC, NW, CAPG, GSZ), jnp.float32),
        jax.ShapeDtypeStruct((NW, 16), jnp.int32),
    ],
    mesh=_mesh,
    scratch_types=[
        pltpu.VMEM((P_G, 2, GSZ), jnp.int32),
        pltpu.VMEM((P_G, GSZ), jnp.float32),
        pltpu.VMEM((STG_E + 16,), jnp.int32),
        pltpu.VMEM((STG_E + 16,), jnp.int32),
        pltpu.VMEM((STG_E + 16,), jnp.float32),
        pltpu.VMEM((STG_E + 16,), jnp.int32),
        pltpu.VMEM((STG_E + 16,), jnp.int32),
        pltpu.VMEM((STG_E + 16,), jnp.float32),
        pltpu.VMEM((CHUNK_G, GSZ), jnp.int32),
        pltpu.VMEM((CHUNK_G, GSZ), jnp.float32),
        pltpu.VMEM((16,), jnp.int32),
        pltpu.SMEM((8,), jnp.int32),
    ],
    compiler_params=_cparams_nl,
)
def _partition(epk, vpk, bR, bC, bV, bcnt,
               ine, inv, sR0, sC0, sV0, sR1, sC1, sV1,
               zR, zV, ctv, sm):
    cid = lax.axis_index("c")
    sid = lax.axis_index("s")
    w = cid * NS + sid

    zi = jnp.zeros((16,), jnp.int32)
    zf = jnp.zeros((16,), jnp.float32)
    it16 = _iota16()

    @pl.loop(0, CHUNK_E // 16)
    def _(i):
        sl = pl.ds(i * 16, 16)
        zV[i // 8, pl.ds((i % 8) * 16, 16)] = zf
        zR[i // 8, pl.ds((i % 8) * 16, 16)] = (it16 + i * 16) & 32767

    sm[4] = 0
    sm[5] = 0

    @pl.loop(0, PASSES)
    def _(p):
        @pl.loop(0, (STG_E + 16) // 16)
        def _(i):
            sl = pl.ds(i * 16, 16)
            rspread = (it16 + i * 16) & 32767
            sR0[sl] = rspread
            sC0[sl] = zi
            sV0[sl] = zf
            sR1[sl] = rspread
            sC1[sl] = zi
            sV1[sl] = zf

        gbase = w * PT_GROUPS + p * P_G
        pltpu.sync_copy(epk.at[pl.ds(gbase, P_G)], ine)
        pltpu.sync_copy(vpk.at[pl.ds(gbase, P_G)], inv)

        sm[0] = 0
        sm[1] = 0
        sm[2] = 0
        sm[3] = 0

        @pl.loop(0, P_G * (GSZ // 16))
        def _(v):
            g = v // (GSZ // 16)
            sl = pl.ds((v % (GSZ // 16)) * 16, 16)
            rv = ine[g, 0, sl]
            cv = ine[g, 1, sl]
            vv = inv[g, sl]
            mA = rv < HALF_N
            nA = jnp.sum(jnp.where(mA, 1, 0))
            cntA = sm[0]
            cntB = sm[1]
            plsc.store_compressed(sR0.at[pl.ds(cntA, 16)], rv, mask=mA)
            plsc.store_compressed(sC0.at[pl.ds(cntA, 16)], cv, mask=mA)
            plsc.store_compressed(sV0.at[pl.ds(cntA, 16)], vv, mask=mA)
            mB = jnp.logical_not(mA)
            plsc.store_compressed(sR1.at[pl.ds(cntB, 16)], rv - HALF_N, mask=mB)
            plsc.store_compressed(sC1.at[pl.ds(cntB, 16)], cv, mask=mB)
            plsc.store_compressed(sV1.at[pl.ds(cntB, 16)], vv, mask=mB)
            sm[0] = cntA + nA
            sm[1] = cntB + (16 - nA)

            @pl.when(sm[0] - sm[2] * GSZ >= GSZ)
            def _():
                fA = sm[2]
                g = sm[4] + fA
                pltpu.sync_copy(sR0.at[pl.ds(fA * GSZ, GSZ)], bR.at[0, w, g])
                pltpu.sync_copy(sC0.at[pl.ds(fA * GSZ, GSZ)], bC.at[0, w, g])
                pltpu.sync_copy(sV0.at[pl.ds(fA * GSZ, GSZ)], bV.at[0, w, g])
                sm[2] = fA + 1

            @pl.when(sm[1] - sm[3] * GSZ >= GSZ)
            def _():
                fB = sm[3]
                g = sm[5] + fB
                pltpu.sync_copy(sR1.at[pl.ds(fB * GSZ, GSZ)], bR.at[1, w, g])
                pltpu.sync_copy(sC1.at[pl.ds(fB * GSZ, GSZ)], bC.at[1, w, g])
                pltpu.sync_copy(sV1.at[pl.ds(fB * GSZ, GSZ)], bV.at[1, w, g])
                sm[3] = fB + 1

        @pl.when(sm[0] > sm[2] * GSZ)
        def _():
            g = sm[4] + sm[2]
            pltpu.sync_copy(sR0.at[pl.ds(sm[2] * GSZ, GSZ)], bR.at[0, w, g])
            pltpu.sync_copy(sC0.at[pl.ds(sm[2] * GSZ, GSZ)], bC.at[0, w, g])
            pltpu.sync_copy(sV0.at[pl.ds(sm[2] * GSZ, GSZ)], bV.at[0, w, g])

        @pl.when(sm[1] > sm[3] * GSZ)
        def _():
            g = sm[5] + sm[3]
            pltpu.sync_copy(sR1.at[pl.ds(sm[3] * GSZ, GSZ)], bR.at[1, w, g])
            pltpu.sync_copy(sC1.at[pl.ds(sm[3] * GSZ, GSZ)], bC.at[1, w, g])
            pltpu.sync_copy(sV1.at[pl.ds(sm[3] * GSZ, GSZ)], bV.at[1, w, g])

        sm[4] = sm[4] + (sm[0] + GSZ - 1) // GSZ
        sm[5] = sm[5] + (sm[1] + GSZ - 1) // GSZ

    gA = sm[4]
    gB = sm[5]
    pltpu.sync_copy(zR, bR.at[0, w, pl.ds(gA, CHUNK_G)])
    pltpu.sync_copy(zR, bR.at[1, w, pl.ds(gB, CHUNK_G)])
    zc = jnp.zeros((16,), jnp.int32)

    @pl.loop(0, CHUNK_E // 16)
    def _(i):
        zR[i // 8, pl.ds((i % 8) * 16, 16)] = zc

    pltpu.sync_copy(zR, bC.at[0, w, pl.ds(gA, CHUNK_G)])
    pltpu.sync_copy(zR, bC.at[1, w, pl.ds(gB, CHUNK_G)])
    pltpu.sync_copy(zV, bV.at[0, w, pl.ds(gA, CHUNK_G)])
    pltpu.sync_copy(zV, bV.at[1, w, pl.ds(gB, CHUNK_G)])

    it = _iota16()
    ctv[pl.ds(0, 16)] = jnp.where(it == 0, gA, jnp.where(it == 1, gB, 0))
    pltpu.sync_copy(ctv, bcnt.at[w])


@functools.partial(
    pl.kernel,
    out_type=jax.ShapeDtypeStruct((N_C, D_C), jnp.bfloat16),
    mesh=_mesh,
    scratch_types=[
        pltpu.VMEM_SHARED((HALF_N, D_C), jnp.float32),
        pltpu.VMEM((CHUNK_G, GSZ), jnp.int32),
        pltpu.VMEM((CHUNK_G, GSZ), jnp.int32),
        pltpu.VMEM((CHUNK_G, GSZ), jnp.float32),
        pltpu.VMEM((GSZ, D_C), jnp.bfloat16),
        pltpu.VMEM((GSZ, D_C), jnp.bfloat16),
        pltpu.VMEM((GSZ, D_C), jnp.bfloat16),
        pltpu.VMEM((GSZ, D_C), jnp.bfloat16),
        pltpu.VMEM((GSZ, D_C), jnp.float32),
        pltpu.VMEM((GSZ, D_C), jnp.float32),
        pltpu.VMEM((ZB, D_C), jnp.float32),
        pltpu.VMEM((ZB, D_C), jnp.bfloat16),
        pltpu.VMEM((16,), jnp.int32),
        pltpu.SemaphoreType.DMA,
        pltpu.SemaphoreType.DMA,
        pltpu.SemaphoreType.DMA,
        pltpu.SemaphoreType.DMA,
        pltpu.SemaphoreType.DMA,
        pltpu.SemaphoreType.DMA,
    ],
    compiler_params=_cparams_nl,
)
def _spmm(ego, bR, bC, bV, bcnt, out,
          acc, rows2, colf, valf, rb0, rb1, rb2, rb3, mb0, mb1,
          wf, wb, ctv, sg0, sg1, sg2, sg3, ss0, ss1):
    cid = lax.axis_index("c")
    sid = lax.axis_index("s")
    rbs = (rb0, rb1, rb2, rb3)
    gsem = (sg0, sg1, sg2, sg3)
    msgs = (mb0, mb1)
    ssem = (ss0, ss1)

    @pl.loop(0, ZB)
    def _(r):
        wf[r, pl.ds(0, 16)] = jnp.zeros((16,), jnp.float32)
        wf[r, pl.ds(16, 16)] = jnp.zeros((16,), jnp.float32)

    base = jnp.minimum(sid * STRIPE, HALF_N - STRIPE)

    @pl.loop(0, (STRIPE + ZB - 1) // ZB)
    def _(i):
        off = jnp.minimum(i * ZB, STRIPE - ZB)
        pltpu.sync_copy(wf, acc.at[pl.ds(base + off, ZB)])

    plsc.subcore_barrier()

    def scale(q, rb, mb):
        @pl.loop(0, GSZ // 16)
        def _(v):
            wv = valf[q, pl.ds(v * 16, 16)]
            for ee in range(16):
                wvec = lax.gather(
                    wv, jnp.full((16, 1), ee, jnp.int32), _GD, (1,),
                    mode=lax.GatherScatterMode.PROMISE_IN_BOUNDS)
                e = v * 16 + ee
                a, b = plsc.unpack(rb[e], format=plsc.PackFormat.INTERLEAVED)
                mb[e, pl.ds(0, 16)] = a * wvec
                mb[e, pl.ds(16, 16)] = b * wvec

    it = _iota16()
    for bi in range(2):
        bkt = sid * 2 + bi
        pltpu.sync_copy(bcnt.at[bkt], ctv)
        n_g = jnp.sum(jnp.where(it == cid, ctv[pl.ds(0, 16)], 0))
        n_chunks = (n_g + CHUNK_G - 1) // CHUNK_G

        @pl.loop(0, MAXC)
        def _(c):
            @pl.when(c < n_chunks)
            def _():
                pltpu.sync_copy(bR.at[cid, bkt, pl.ds(c * CHUNK_G, CHUNK_G)],
                                rows2)
                pltpu.sync_copy(bC.at[cid, bkt, pl.ds(c * CHUNK_G, CHUNK_G)],
                                colf)
                pltpu.sync_copy(bV.at[cid, bkt, pl.ds(c * CHUNK_G, CHUNK_G)],
                                valf)

                pltpu.async_copy(ego.at[colf.at[0]], rbs[0], gsem[0])
                pltpu.async_copy(ego.at[colf.at[1]], rbs[1], gsem[1])

                @pl.loop(0, CHUNK_G // 4)
                def _(i):
                    for k in range(4):
                        q = 4 * i + k
                        nxt = (k + 2) % 4
                        m = k % 2
                        if k < 2:
                            @pl.when(i > 0)
                            def _():
                                pltpu.make_async_copy(
                                    msgs[m], acc.at[rows2.at[0]],
                                    ssem[m]).wait()
                        else:
                            pltpu.make_async_copy(
                                msgs[m], acc.at[rows2.at[0]], ssem[m]).wait()

                        @pl.when(q + 2 < CHUNK_G)
                        def _():
                            pltpu.async_copy(
                                ego.at[colf.at[q + 2]], rbs[nxt], gsem[nxt])

                        pltpu.make_async_copy(
                            ego.at[colf.at[0]], rbs[k], gsem[k]).wait()
                        scale(q, rbs[k], msgs[m])
                        pltpu.async_copy(
                            msgs[m], acc.at[rows2.at[q]], ssem[m], add=True)

                pltpu.make_async_copy(
                    msgs[0], acc.at[rows2.at[0]], ssem[0]).wait()
                pltpu.make_async_copy(
                    msgs[1], acc.at[rows2.at[0]], ssem[1]).wait()

    plsc.subcore_barrier()

    @pl.loop(0, (STRIPE + ZB - 1) // ZB)
    def _(i):
        off = base + jnp.minimum(i * ZB, STRIPE - ZB)
        pltpu.sync_copy(acc.at[pl.ds(off, ZB)], wf)

        @pl.loop(0, ZB)
        def _(r):
            a = wf[r, pl.ds(0, 16)]
            b = wf[r, pl.ds(16, 16)]
            wb[r] = plsc.pack(a, b, format=plsc.PackFormat.INTERLEAVED)

        pltpu.sync_copy(wb, out.at[pl.ds(cid * HALF_N + off, ZB)])


IDX_TOTAL = 3 * B_C
IDX_G = IDX_TOTAL // GSZ
IDX_G_PER_TILE = 8
IDX_TILES = IDX_G // IDX_G_PER_TILE


@functools.partial(
    pl.kernel,
    out_type=[jax.ShapeDtypeStruct((IDX_TOTAL, D_C), jnp.bfloat16)] * (LAYERS_C + 1),
    mesh=_mesh,
    scratch_types=[
        pltpu.VMEM((IDX_G_PER_TILE, GSZ), jnp.int32),
        pltpu.VMEM((GSZ, D_C), jnp.bfloat16),
    ],
    compiler_params=_cparams,
)
def _gather4(t0, t1, t2, t3, idx_hbm, o0, o1, o2, o3, idxv, buf):
    cid = lax.axis_index("c")
    sid = lax.axis_index("s")
    w = cid * NS + sid

    @pl.when(w < IDX_TILES)
    def _():
        pltpu.sync_copy(
            idx_hbm.at[pl.ds(w * IDX_G_PER_TILE, IDX_G_PER_TILE)], idxv)
        for tab, outb in ((t0, o0), (t1, o1), (t2, o2), (t3, o3)):
            @pl.loop(0, IDX_G_PER_TILE)
            def _(j):
                pltpu.sync_copy(tab.at[idxv.at[j]], buf)
                pltpu.sync_copy(
                    buf, outb.at[pl.ds((w * IDX_G_PER_TILE + j) * GSZ, GSZ)])


def kernel(users, pos_items, neg_items, edge_index, adj_vals, user_emb, item_emb):
    ego0 = jnp.concatenate([user_emb, item_emb], axis=0)
    ego0_bf = ego0.astype(jnp.bfloat16)
    row = edge_index[0].astype(jnp.int32)
    col = edge_index[1].astype(jnp.int32)
    pad = E_PAD - E_C
    rowp = jnp.concatenate(
        [row, jnp.arange(pad, dtype=jnp.int32) & 32767]).reshape(G_TOTAL, GSZ)
    colp = jnp.pad(col, (0, pad)).reshape(G_TOTAL, GSZ)
    vpk = jnp.pad(adj_vals, (0, pad)).reshape(G_TOTAL, GSZ)
    epk = jnp.stack([rowp, colp], axis=1)

    bR, bC, bV, bcnt = _partition(epk, vpk)

    tabs = [ego0_bf]
    for _ in range(LAYERS_C):
        tabs.append(_spmm(tabs[-1], bR, bC, bV, bcnt))

    idx_all = jnp.concatenate([
        users.astype(jnp.int32),
        pos_items.astype(jnp.int32) + N_USER_C,
        neg_items.astype(jnp.int32) + N_USER_C,
    ]).reshape(IDX_G, GSZ)

    g = _gather4(tabs[0], tabs[1], tabs[2], tabs[3], idx_all)
    cat = jnp.concatenate(g, axis=1).astype(jnp.float32)
    return (cat[:B_C], cat[B_C:2 * B_C], cat[2 * B_C:])


# speedup vs baseline: 1.5181x; 1.5181x over previous
import functools

import jax
import jax.numpy as jnp
from jax import lax
from jax.experimental import pallas as pl
from jax.experimental.pallas import tpu as pltpu
from jax.experimental.pallas import tpu_sc as plsc

N_USER_C = 50000
N_ITEM_C = 50000
N_C = N_USER_C + N_ITEM_C
E_C = 1600000
D_C = 32
HD = 16
B_C = 4096
LAYERS_C = 3

NC = 2
NS = 16

GSZ = 128
GROUPS_PER_TILE = 800
E_PAD = NS * GROUPS_PER_TILE * GSZ
G_TOTAL = E_PAD // GSZ
CHUNK_G = 20
CHUNKS = GROUPS_PER_TILE // CHUNK_G

STRIPE = 6256
N_ZERO = (STRIPE + GSZ - 1) // GSZ

_mesh = plsc.VectorSubcoreMesh(core_axis_name="c", subcore_axis_name="s")
_cparams = pltpu.CompilerParams(use_tc_tiling_on_sc=False)


def _scale(vbuf, jj, rb):
    @pl.loop(0, GSZ // 16)
    def _(v):
        wv = vbuf[jj, pl.ds(v * 16, 16)]
        for ee in range(16):
            w = jnp.broadcast_to(wv[ee:ee + 1], (16,))
            e = v * 16 + ee
            rb[e, pl.ds(0, HD)] = rb[e, pl.ds(0, HD)] * w


def _layer(sid, ego, out, epk, vpk, acc, eb0, eb1, vb0, vb1,
           rbs, zbuf, sem_e, gsem, ssem):
    @pl.loop(0, GSZ)
    def _(e):
        zbuf[e, pl.ds(0, HD)] = jnp.zeros((HD,), jnp.float32)

    base = jnp.minimum(sid * STRIPE, N_C - STRIPE)

    @pl.loop(0, N_ZERO)
    def _(g):
        off = jnp.minimum(g * GSZ, STRIPE - GSZ)
        pltpu.sync_copy(zbuf, acc.at[pl.ds(base + off, GSZ)])

    plsc.subcore_barrier()

    gb0 = sid * GROUPS_PER_TILE
    pltpu.async_copy(epk.at[pl.ds(gb0, CHUNK_G)], eb0, sem_e)
    pltpu.async_copy(vpk.at[pl.ds(gb0, CHUNK_G)], vb0, sem_e)

    def do_chunk(c, ebuf, ebnext, vbuf, vbnext):
        def scat_wait(b):
            pltpu.make_async_copy(
                rbs[b], acc.at[ebuf.at[0, 0]], ssem[b]).wait()

        pltpu.make_async_copy(epk.at[pl.ds(0, CHUNK_G)], ebuf, sem_e).wait()
        pltpu.make_async_copy(vpk.at[pl.ds(0, CHUNK_G)], vbuf, sem_e).wait()

        @pl.when(c + 1 < CHUNKS)
        def _():
            nb = gb0 + (c + 1) * CHUNK_G
            pltpu.async_copy(epk.at[pl.ds(nb, CHUNK_G)], ebnext, sem_e)
            pltpu.async_copy(vpk.at[pl.ds(nb, CHUNK_G)], vbnext, sem_e)

        pltpu.async_copy(ego.at[ebuf.at[0, 1]], rbs[0], gsem[0])
        pltpu.async_copy(ego.at[ebuf.at[1, 1]], rbs[1], gsem[1])

        @pl.loop(0, CHUNK_G // 4)
        def _(i):
            for k in range(4):
                q = 4 * i + k
                nxt = (k + 2) % 4
                prv = (k + 3) % 4
                if k == 0:
                    @pl.when(i > 0)
                    def _():
                        scat_wait(prv)
                else:
                    scat_wait(prv)

                @pl.when(q + 2 < CHUNK_G)
                def _():
                    pltpu.async_copy(
                        ego.at[ebuf.at[q + 2, 1]], rbs[nxt], gsem[nxt])

                pltpu.make_async_copy(
                    ego.at[pl.ds(0, GSZ)], rbs[k], gsem[k]).wait()
                _scale(vbuf, q, rbs[k])
                pltpu.async_copy(
                    rbs[k], acc.at[ebuf.at[q, 0]], ssem[k], add=True)

        scat_wait(3)

    @pl.loop(0, CHUNKS // 2)
    def _(c2):
        do_chunk(2 * c2, eb0, eb1, vb0, vb1)
        do_chunk(2 * c2 + 1, eb1, eb0, vb1, vb0)

    plsc.subcore_barrier()
    pltpu.sync_copy(acc.at[pl.ds(base, STRIPE)], out.at[pl.ds(base, STRIPE)])


@functools.partial(
    pl.kernel,
    out_type=[jax.ShapeDtypeStruct((N_C, HD), jnp.float32)] * (2 * LAYERS_C),
    mesh=_mesh,
    scratch_types=[
        pltpu.VMEM_SHARED((N_C, HD), jnp.float32),
        pltpu.VMEM((CHUNK_G, 2, GSZ), jnp.int32),
        pltpu.VMEM((CHUNK_G, 2, GSZ), jnp.int32),
        pltpu.VMEM((CHUNK_G, GSZ), jnp.float32),
        pltpu.VMEM((CHUNK_G, GSZ), jnp.float32),
        pltpu.VMEM((GSZ, HD), jnp.float32),
        pltpu.VMEM((GSZ, HD), jnp.float32),
        pltpu.VMEM((GSZ, HD), jnp.float32),
        pltpu.VMEM((GSZ, HD), jnp.float32),
        pltpu.VMEM((GSZ, HD), jnp.float32),
        pltpu.SemaphoreType.DMA,
        pltpu.SemaphoreType.DMA,
        pltpu.SemaphoreType.DMA,
        pltpu.SemaphoreType.DMA,
        pltpu.SemaphoreType.DMA,
        pltpu.SemaphoreType.DMA,
        pltpu.SemaphoreType.DMA,
        pltpu.SemaphoreType.DMA,
        pltpu.SemaphoreType.DMA,
    ],
    compiler_params=_cparams,
)
def _spmm3(egoA, egoB, epk, vpk, oA1, oB1, oA2, oB2, oA3, oB3,
           acc, eb0, eb1, vb0, vb1, rb0, rb1, rb2, rb3, zbuf, sem_e,
           sg0, sg1, sg2, sg3, ss0, ss1, ss2, ss3):
    cid = lax.axis_index("c")
    sid = lax.axis_index("s")
    rbs = (rb0, rb1, rb2, rb3)
    gsem = (sg0, sg1, sg2, sg3)
    ssem = (ss0, ss1, ss2, ss3)

    @pl.when(cid == 0)
    def _():
        for src_t, dst_t in ((egoA, oA1), (oA1, oA2), (oA2, oA3)):
            _layer(sid, src_t, dst_t, epk, vpk, acc, eb0, eb1, vb0, vb1,
                   rbs, zbuf, sem_e, gsem, ssem)
            plsc.subcore_barrier()

    @pl.when(cid == 1)
    def _():
        for src_t, dst_t in ((egoB, oB1), (oB1, oB2), (oB2, oB3)):
            _layer(sid, src_t, dst_t, epk, vpk, acc, eb0, eb1, vb0, vb1,
                   rbs, zbuf, sem_e, gsem, ssem)
            plsc.subcore_barrier()


IDX_TOTAL = 3 * B_C
IDX_G = IDX_TOTAL // GSZ
IDX_G_PER_TILE = 8
IDX_TILES = IDX_G // IDX_G_PER_TILE


@functools.partial(
    pl.kernel,
    out_type=[jax.ShapeDtypeStruct((IDX_TOTAL, HD), jnp.float32)] * (2 * (LAYERS_C + 1)),
    mesh=_mesh,
    scratch_types=[
        pltpu.VMEM((IDX_G_PER_TILE, GSZ), jnp.int32),
        pltpu.VMEM((GSZ, HD), jnp.float32),
    ],
    compiler_params=_cparams,
)
def _gather8(t0, t1, t2, t3, t4, t5, t6, t7, idx_hbm,
             o0, o1, o2, o3, o4, o5, o6, o7, idxv, buf):
    cid = lax.axis_index("c")
    sid = lax.axis_index("s")
    w = cid * NS + sid

    @pl.when(w < IDX_TILES)
    def _():
        pltpu.sync_copy(
            idx_hbm.at[pl.ds(w * IDX_G_PER_TILE, IDX_G_PER_TILE)], idxv)
        for tab, out in ((t0, o0), (t1, o1), (t2, o2), (t3, o3),
                         (t4, o4), (t5, o5), (t6, o6), (t7, o7)):
            @pl.loop(0, IDX_G_PER_TILE)
            def _(j):
                pltpu.sync_copy(tab.at[idxv.at[j]], buf)
                pltpu.sync_copy(
                    buf, out.at[pl.ds((w * IDX_G_PER_TILE + j) * GSZ, GSZ)])


def kernel(users, pos_items, neg_items, edge_index, adj_vals, user_emb, item_emb):
    ego0 = jnp.concatenate([user_emb, item_emb], axis=0)
    row = edge_index[0].astype(jnp.int32)
    col = edge_index[1].astype(jnp.int32)
    pad = E_PAD - E_C
    rowp = jnp.pad(row, (0, pad)).reshape(G_TOTAL, GSZ)
    colp = jnp.pad(col, (0, pad)).reshape(G_TOTAL, GSZ)
    vpk = jnp.pad(adj_vals, (0, pad)).reshape(G_TOTAL, GSZ)
    epk = jnp.stack([rowp, colp], axis=1)

    egoA, egoB = ego0[:, :HD], ego0[:, HD:]
    oA1, oB1, oA2, oB2, oA3, oB3 = _spmm3(egoA, egoB, epk, vpk)
    halves = [(egoA, egoB), (oA1, oB1), (oA2, oB2), (oA3, oB3)]

    idx_all = jnp.concatenate([
        users.astype(jnp.int32),
        pos_items.astype(jnp.int32) + N_USER_C,
        neg_items.astype(jnp.int32) + N_USER_C,
    ]).reshape(IDX_G, GSZ)

    tabs = [h for pair in halves for h in pair]
    g = _gather8(*tabs, idx_all)
    cat = jnp.concatenate(
        [jnp.concatenate([g[2 * k], g[2 * k + 1]], axis=1)
         for k in range(LAYERS_C + 1)], axis=1)
    return (cat[:B_C], cat[B_C:2 * B_C], cat[2 * B_C:])


# speedup vs baseline: 1.6054x; 1.0575x over previous
import functools

import jax
import jax.numpy as jnp
from jax import lax
from jax.experimental import pallas as pl
from jax.experimental.pallas import tpu as pltpu
from jax.experimental.pallas import tpu_sc as plsc

N_USER_C = 50000
N_ITEM_C = 50000
N_C = N_USER_C + N_ITEM_C
E_C = 1600000
D_C = 32
HD = 16
B_C = 4096
LAYERS_C = 3

NC = 2
NS = 16

GSZ = 128
GROUPS_PER_TILE = 800
E_PAD = NS * GROUPS_PER_TILE * GSZ
G_TOTAL = E_PAD // GSZ
CHUNK_G = 20
CHUNKS = GROUPS_PER_TILE // CHUNK_G

STRIPE = 6256
N_ZERO = (STRIPE + GSZ - 1) // GSZ

_mesh = plsc.VectorSubcoreMesh(core_axis_name="c", subcore_axis_name="s")
_cparams = pltpu.CompilerParams(use_tc_tiling_on_sc=False)


def _scale(vbuf, jj, rb):
    @pl.loop(0, GSZ // 16)
    def _(v):
        wv = vbuf[jj, pl.ds(v * 16, 16)]
        for ee in range(16):
            w = jnp.broadcast_to(wv[ee:ee + 1], (16,))
            e = v * 16 + ee
            rb[e, pl.ds(0, HD)] = rb[e, pl.ds(0, HD)] * w


def _layer(sid, ego, out, epk, vpk, acc, eb0, eb1, vb0, vb1,
           rbs, zbuf, sem_e, gsem, ssem):
    @pl.loop(0, GSZ)
    def _(e):
        zbuf[e, pl.ds(0, HD)] = jnp.zeros((HD,), jnp.float32)

    base = jnp.minimum(sid * STRIPE, N_C - STRIPE)

    @pl.loop(0, N_ZERO)
    def _(g):
        off = jnp.minimum(g * GSZ, STRIPE - GSZ)
        pltpu.sync_copy(zbuf, acc.at[pl.ds(base + off, GSZ)])

    plsc.subcore_barrier()

    gb0 = sid * GROUPS_PER_TILE

    def scat_wait(b):
        pltpu.make_async_copy(rbs[b], acc.at[eb0.at[0, 0]], ssem[b]).wait()

    pltpu.async_copy(epk.at[pl.ds(gb0, CHUNK_G)], eb0, sem_e)
    pltpu.async_copy(vpk.at[pl.ds(gb0, CHUNK_G)], vb0, sem_e)
    pltpu.make_async_copy(epk.at[pl.ds(0, CHUNK_G)], eb0, sem_e).wait()
    pltpu.make_async_copy(vpk.at[pl.ds(0, CHUNK_G)], vb0, sem_e).wait()
    pltpu.async_copy(ego.at[eb0.at[0, 1]], rbs[0], gsem[0])
    pltpu.async_copy(ego.at[eb0.at[1, 1]], rbs[1], gsem[1])

    def do_chunk(c, ebuf, ebnext, vbuf, vbnext):
        @pl.loop(0, CHUNK_G // 4)
        def _(i):
            for k in range(4):
                q = 4 * i + k
                nxt = (k + 2) % 4
                prv = (k + 3) % 4
                if k == 0:
                    @pl.when(jnp.logical_or(c > 0, i > 0))
                    def _():
                        scat_wait(prv)
                else:
                    scat_wait(prv)

                if k == 2:
                    @pl.when(jnp.logical_and(i == 0, c + 1 < CHUNKS))
                    def _():
                        nb = gb0 + (c + 1) * CHUNK_G
                        pltpu.async_copy(
                            epk.at[pl.ds(nb, CHUNK_G)], ebnext, sem_e)
                        pltpu.async_copy(
                            vpk.at[pl.ds(nb, CHUNK_G)], vbnext, sem_e)

                @pl.when(q + 2 < CHUNK_G)
                def _():
                    pltpu.async_copy(
                        ego.at[ebuf.at[q + 2, 1]], rbs[nxt], gsem[nxt])

                if k >= 2:
                    @pl.when(jnp.logical_and(q + 2 >= CHUNK_G,
                                             c + 1 < CHUNKS))
                    def _():
                        if k == 2:
                            pltpu.make_async_copy(
                                epk.at[pl.ds(0, CHUNK_G)], ebnext,
                                sem_e).wait()
                            pltpu.make_async_copy(
                                vpk.at[pl.ds(0, CHUNK_G)], vbnext,
                                sem_e).wait()
                        pltpu.async_copy(
                            ego.at[ebnext.at[k - 2, 1]], rbs[nxt],
                            gsem[nxt])

                pltpu.make_async_copy(
                    ego.at[pl.ds(0, GSZ)], rbs[k], gsem[k]).wait()
                _scale(vbuf, q, rbs[k])
                pltpu.async_copy(
                    rbs[k], acc.at[ebuf.at[q, 0]], ssem[k], add=True)

    @pl.loop(0, CHUNKS // 2)
    def _(c2):
        do_chunk(2 * c2, eb0, eb1, vb0, vb1)
        do_chunk(2 * c2 + 1, eb1, eb0, vb1, vb0)

    scat_wait(3)

    plsc.subcore_barrier()
    pltpu.sync_copy(acc.at[pl.ds(base, STRIPE)], out.at[pl.ds(base, STRIPE)])


@functools.partial(
    pl.kernel,
    out_type=[jax.ShapeDtypeStruct((N_C, HD), jnp.float32)] * (2 * LAYERS_C),
    mesh=_mesh,
    scratch_types=[
        pltpu.VMEM_SHARED((N_C, HD), jnp.float32),
        pltpu.VMEM((CHUNK_G, 2, GSZ), jnp.int32),
        pltpu.VMEM((CHUNK_G, 2, GSZ), jnp.int32),
        pltpu.VMEM((CHUNK_G, GSZ), jnp.float32),
        pltpu.VMEM((CHUNK_G, GSZ), jnp.float32),
        pltpu.VMEM((GSZ, HD), jnp.float32),
        pltpu.VMEM((GSZ, HD), jnp.float32),
        pltpu.VMEM((GSZ, HD), jnp.float32),
        pltpu.VMEM((GSZ, HD), jnp.float32),
        pltpu.VMEM((GSZ, HD), jnp.float32),
        pltpu.SemaphoreType.DMA,
        pltpu.SemaphoreType.DMA,
        pltpu.SemaphoreType.DMA,
        pltpu.SemaphoreType.DMA,
        pltpu.SemaphoreType.DMA,
        pltpu.SemaphoreType.DMA,
        pltpu.SemaphoreType.DMA,
        pltpu.SemaphoreType.DMA,
        pltpu.SemaphoreType.DMA,
    ],
    compiler_params=_cparams,
)
def _spmm3(egoA, egoB, epk, vpk, oA1, oB1, oA2, oB2, oA3, oB3,
           acc, eb0, eb1, vb0, vb1, rb0, rb1, rb2, rb3, zbuf, sem_e,
           sg0, sg1, sg2, sg3, ss0, ss1, ss2, ss3):
    cid = lax.axis_index("c")
    sid = lax.axis_index("s")
    rbs = (rb0, rb1, rb2, rb3)
    gsem = (sg0, sg1, sg2, sg3)
    ssem = (ss0, ss1, ss2, ss3)

    @pl.when(cid == 0)
    def _():
        for src_t, dst_t in ((egoA, oA1), (oA1, oA2), (oA2, oA3)):
            _layer(sid, src_t, dst_t, epk, vpk, acc, eb0, eb1, vb0, vb1,
                   rbs, zbuf, sem_e, gsem, ssem)
            plsc.subcore_barrier()

    @pl.when(cid == 1)
    def _():
        for src_t, dst_t in ((egoB, oB1), (oB1, oB2), (oB2, oB3)):
            _layer(sid, src_t, dst_t, epk, vpk, acc, eb0, eb1, vb0, vb1,
                   rbs, zbuf, sem_e, gsem, ssem)
            plsc.subcore_barrier()


IDX_TOTAL = 3 * B_C
IDX_G = IDX_TOTAL // GSZ
IDX_G_PER_TILE = 8
IDX_TILES = IDX_G // IDX_G_PER_TILE


@functools.partial(
    pl.kernel,
    out_type=[jax.ShapeDtypeStruct((IDX_TOTAL, HD), jnp.float32)] * (2 * (LAYERS_C + 1)),
    mesh=_mesh,
    scratch_types=[
        pltpu.VMEM((IDX_G_PER_TILE, GSZ), jnp.int32),
        pltpu.VMEM((GSZ, HD), jnp.float32),
    ],
    compiler_params=_cparams,
)
def _gather8(t0, t1, t2, t3, t4, t5, t6, t7, idx_hbm,
             o0, o1, o2, o3, o4, o5, o6, o7, idxv, buf):
    cid = lax.axis_index("c")
    sid = lax.axis_index("s")
    w = cid * NS + sid

    @pl.when(w < IDX_TILES)
    def _():
        pltpu.sync_copy(
            idx_hbm.at[pl.ds(w * IDX_G_PER_TILE, IDX_G_PER_TILE)], idxv)
        for tab, out in ((t0, o0), (t1, o1), (t2, o2), (t3, o3),
                         (t4, o4), (t5, o5), (t6, o6), (t7, o7)):
            @pl.loop(0, IDX_G_PER_TILE)
            def _(j):
                pltpu.sync_copy(tab.at[idxv.at[j]], buf)
                pltpu.sync_copy(
                    buf, out.at[pl.ds((w * IDX_G_PER_TILE + j) * GSZ, GSZ)])


def kernel(users, pos_items, neg_items, edge_index, adj_vals, user_emb, item_emb):
    ego0 = jnp.concatenate([user_emb, item_emb], axis=0)
    row = edge_index[0].astype(jnp.int32)
    col = edge_index[1].astype(jnp.int32)
    pad = E_PAD - E_C
    rowp = jnp.pad(row, (0, pad)).reshape(G_TOTAL, GSZ)
    colp = jnp.pad(col, (0, pad)).reshape(G_TOTAL, GSZ)
    vpk = jnp.pad(adj_vals, (0, pad)).reshape(G_TOTAL, GSZ)
    epk = jnp.stack([rowp, colp], axis=1)

    egoA, egoB = ego0[:, :HD], ego0[:, HD:]
    oA1, oB1, oA2, oB2, oA3, oB3 = _spmm3(egoA, egoB, epk, vpk)
    halves = [(egoA, egoB), (oA1, oB1), (oA2, oB2), (oA3, oB3)]

    idx_all = jnp.concatenate([
        users.astype(jnp.int32),
        pos_items.astype(jnp.int32) + N_USER_C,
        neg_items.astype(jnp.int32) + N_USER_C,
    ]).reshape(IDX_G, GSZ)

    tabs = [h for pair in halves for h in pair]
    g = _gather8(*tabs, idx_all)
    cat = jnp.concatenate(
        [jnp.concatenate([g[2 * k], g[2 * k + 1]], axis=1)
         for k in range(LAYERS_C + 1)], axis=1)
    return (cat[:B_C], cat[B_C:2 * B_C], cat[2 * B_C:])
